# baseline (speedup 1.0000x reference)
import functools

import jax
import jax.numpy as jnp
from jax import lax
from jax.experimental import pallas as pl
from jax.experimental.pallas import tpu as pltpu
from jax.experimental.pallas import tpu_sc as plsc

EMB = 32
K = 8192
TOK = 16384
T = 1024
KC = 2048
COMMIT = 0.25


def _argmin_body(x_ref, wt_ref, ind_ref, acc_ref):
    x = x_ref[...]
    x2 = jnp.sum(x * x, axis=1, keepdims=True)
    run_m = jnp.full((T, 1), jnp.inf, jnp.float32)
    run_a = jnp.zeros((T, 1), jnp.int32)
    for c in range(K // KC):
        wt_c = wt_ref[:, c * KC:(c + 1) * KC]
        w2 = jnp.sum(wt_c * wt_c, axis=0, keepdims=True)
        d = x2 - 2.0 * jnp.dot(x, wt_c, preferred_element_type=jnp.float32) + w2
        m = jnp.min(d, axis=1, keepdims=True)
        a = jnp.argmin(d, axis=1).reshape(T, 1).astype(jnp.int32) + c * KC
        better = m < run_m
        run_a = jnp.where(better, a, run_a)
        run_m = jnp.where(better, m, run_m)
    ind_ref[...] = run_a
    @pl.when(pl.program_id(0) == 0)
    def _():
        acc_ref[0, 0] = 0.0
    acc_ref[0, 0] += jnp.sum(run_m)


def _argmin_call(flat, wt):
    return pl.pallas_call(
        _argmin_body,
        grid=(TOK // T,),
        in_specs=[
            pl.BlockSpec((T, EMB), lambda i: (i, 0)),
            pl.BlockSpec((EMB, K), lambda i: (0, 0)),
        ],
        out_specs=[
            pl.BlockSpec((T, 1), lambda i: (i, 0)),
            pl.BlockSpec(memory_space=pltpu.SMEM),
        ],
        out_shape=[
            jax.ShapeDtypeStruct((TOK, 1), jnp.int32),
            jax.ShapeDtypeStruct((1, 1), jnp.float32),
        ],
    )(flat, wt)


@functools.cache
def _gather_call():
    info = plsc.get_sparse_core_info()
    nc, ns = info.num_cores, info.num_subcores
    bpw = TOK // (nc * ns)

    @functools.partial(
        pl.kernel,
        mesh=plsc.VectorSubcoreMesh(core_axis_name="c", subcore_axis_name="s"),
        compiler_params=pltpu.CompilerParams(use_tc_tiling_on_sc=False),
        out_type=jax.ShapeDtypeStruct((TOK, EMB), jnp.float32),
        scratch_types=[
            pltpu.VMEM((bpw,), jnp.int32),
            pltpu.VMEM((bpw, EMB), jnp.float32),
            pltpu.SemaphoreType.DMA,
        ],
    )
    def sc_gather(table_hbm, idx_hbm, out_hbm, idx_v, rows_v, sem):
        wid = lax.axis_index("s") * nc + lax.axis_index("c")
        base = wid * bpw
        pltpu.sync_copy(idx_hbm.at[pl.ds(base, bpw)], idx_v)
        pltpu.async_copy(table_hbm.at[idx_v], rows_v, sem).wait()
        pltpu.sync_copy(rows_v, out_hbm.at[pl.ds(base, bpw)])

    return sc_gather


def kernel(input, W):
    x = jnp.swapaxes(input, 1, -1)
    flat = x.reshape(TOK, EMB)
    ind2, acc = _argmin_call(flat, W.T)
    ind = ind2.reshape(TOK)
    q = _gather_call()(W, ind)
    diff = (1.0 + COMMIT) * acc[0, 0] / (TOK * EMB)
    quantize = jnp.swapaxes(q.reshape(x.shape), 1, -1)
    return quantize, diff, ind.reshape(x.shape[:-1])


# speedup vs baseline: 1.0979x; 1.0979x over previous
import functools

import jax
import jax.numpy as jnp
from jax import lax
from jax.experimental import pallas as pl
from jax.experimental.pallas import tpu as pltpu
from jax.experimental.pallas import tpu_sc as plsc

EMB = 32
K = 8192
TOK = 16384
T = 512
KC = 4096
COMMIT = 0.25


def _argmin_body(x_ref, wt_ref, ind_ref, acc_ref):
    x = x_ref[...]
    x2 = jnp.sum(x * x, axis=1, keepdims=True)
    run_m = jnp.full((T, 1), jnp.inf, jnp.float32)
    run_a = jnp.zeros((T, 1), jnp.int32)
    for c in range(K // KC):
        wt_c = wt_ref[:, c * KC:(c + 1) * KC]
        w2 = jnp.sum(wt_c * wt_c, axis=0, keepdims=True)
        d = x2 - 2.0 * jnp.dot(x, wt_c, preferred_element_type=jnp.float32) + w2
        m = jnp.min(d, axis=1, keepdims=True)
        a = jnp.argmin(d, axis=1).reshape(T, 1).astype(jnp.int32) + c * KC
        better = m < run_m
        run_a = jnp.where(better, a, run_a)
        run_m = jnp.where(better, m, run_m)
    ind_ref[...] = run_a
    @pl.when(pl.program_id(0) == 0)
    def _():
        acc_ref[0, 0] = 0.0
    acc_ref[0, 0] += jnp.sum(run_m)


def _argmin_call(flat, wt):
    return pl.pallas_call(
        _argmin_body,
        grid=(TOK // T,),
        in_specs=[
            pl.BlockSpec((T, EMB), lambda i: (i, 0)),
            pl.BlockSpec((EMB, K), lambda i: (0, 0)),
        ],
        out_specs=[
            pl.BlockSpec((T, 1), lambda i: (i, 0)),
            pl.BlockSpec(memory_space=pltpu.SMEM),
        ],
        out_shape=[
            jax.ShapeDtypeStruct((TOK, 1), jnp.int32),
            jax.ShapeDtypeStruct((1, 1), jnp.float32),
        ],
    )(flat, wt)


@functools.cache
def _gather_call():
    info = plsc.get_sparse_core_info()
    nc, ns = info.num_cores, info.num_subcores
    bpw = TOK // (nc * ns)

    @functools.partial(
        pl.kernel,
        mesh=plsc.VectorSubcoreMesh(core_axis_name="c", subcore_axis_name="s"),
        compiler_params=pltpu.CompilerParams(use_tc_tiling_on_sc=False),
        out_type=jax.ShapeDtypeStruct((TOK, EMB), jnp.float32),
        scratch_types=[
            pltpu.VMEM((bpw,), jnp.int32),
            pltpu.VMEM((bpw, EMB), jnp.float32),
            pltpu.SemaphoreType.DMA,
        ],
    )
    def sc_gather(table_hbm, idx_hbm, out_hbm, idx_v, rows_v, sem):
        wid = lax.axis_index("s") * nc + lax.axis_index("c")
        base = wid * bpw
        pltpu.sync_copy(idx_hbm.at[pl.ds(base, bpw)], idx_v)
        pltpu.async_copy(table_hbm.at[idx_v], rows_v, sem).wait()
        pltpu.sync_copy(rows_v, out_hbm.at[pl.ds(base, bpw)])

    return sc_gather


def kernel(input, W):
    x = jnp.swapaxes(input, 1, -1)
    flat = x.reshape(TOK, EMB)
    ind2, acc = _argmin_call(flat, W.T)
    ind = ind2.reshape(TOK)
    q = _gather_call()(W, ind)
    diff = (1.0 + COMMIT) * acc[0, 0] / (TOK * EMB)
    quantize = jnp.swapaxes(q.reshape(x.shape), 1, -1)
    return quantize, diff, ind.reshape(x.shape[:-1])


# speedup vs baseline: 1.1038x; 1.0053x over previous
import functools

import jax
import jax.numpy as jnp
from jax import lax
from jax.experimental import pallas as pl
from jax.experimental.pallas import tpu as pltpu
from jax.experimental.pallas import tpu_sc as plsc

EMB = 32
K = 8192
TOK = 16384
T = 512
KC = 8192
COMMIT = 0.25


def _argmin_body(x_ref, wt_ref, ind_ref, acc_ref):
    x = x_ref[...]
    x2 = jnp.sum(x * x, axis=1, keepdims=True)
    run_m = jnp.full((T, 1), jnp.inf, jnp.float32)
    run_a = jnp.zeros((T, 1), jnp.int32)
    for c in range(K // KC):
        wt_c = wt_ref[:, c * KC:(c + 1) * KC]
        w2 = jnp.sum(wt_c * wt_c, axis=0, keepdims=True)
        d = x2 - 2.0 * jnp.dot(x, wt_c, preferred_element_type=jnp.float32) + w2
        m = jnp.min(d, axis=1, keepdims=True)
        a = jnp.argmin(d, axis=1).reshape(T, 1).astype(jnp.int32) + c * KC
        better = m < run_m
        run_a = jnp.where(better, a, run_a)
        run_m = jnp.where(better, m, run_m)
    ind_ref[...] = run_a
    @pl.when(pl.program_id(0) == 0)
    def _():
        acc_ref[0, 0] = 0.0
    acc_ref[0, 0] += jnp.sum(run_m)


def _argmin_call(flat, wt):
    return pl.pallas_call(
        _argmin_body,
        grid=(TOK // T,),
        in_specs=[
            pl.BlockSpec((T, EMB), lambda i: (i, 0)),
            pl.BlockSpec((EMB, K), lambda i: (0, 0)),
        ],
        out_specs=[
            pl.BlockSpec((T, 1), lambda i: (i, 0)),
            pl.BlockSpec(memory_space=pltpu.SMEM),
        ],
        out_shape=[
            jax.ShapeDtypeStruct((TOK, 1), jnp.int32),
            jax.ShapeDtypeStruct((1, 1), jnp.float32),
        ],
    )(flat, wt)


@functools.cache
def _gather_call():
    info = plsc.get_sparse_core_info()
    nc, ns = info.num_cores, info.num_subcores
    bpw = TOK // (nc * ns)

    @functools.partial(
        pl.kernel,
        mesh=plsc.VectorSubcoreMesh(core_axis_name="c", subcore_axis_name="s"),
        compiler_params=pltpu.CompilerParams(use_tc_tiling_on_sc=False),
        out_type=jax.ShapeDtypeStruct((TOK, EMB), jnp.float32),
        scratch_types=[
            pltpu.VMEM((bpw,), jnp.int32),
            pltpu.VMEM((bpw, EMB), jnp.float32),
            pltpu.SemaphoreType.DMA,
        ],
    )
    def sc_gather(table_hbm, idx_hbm, out_hbm, idx_v, rows_v, sem):
        wid = lax.axis_index("s") * nc + lax.axis_index("c")
        base = wid * bpw
        pltpu.sync_copy(idx_hbm.at[pl.ds(base, bpw)], idx_v)
        pltpu.async_copy(table_hbm.at[idx_v], rows_v, sem).wait()
        pltpu.sync_copy(rows_v, out_hbm.at[pl.ds(base, bpw)])

    return sc_gather


def kernel(input, W):
    x = jnp.swapaxes(input, 1, -1)
    flat = x.reshape(TOK, EMB)
    ind2, acc = _argmin_call(flat, W.T)
    ind = ind2.reshape(TOK)
    q = _gather_call()(W, ind)
    diff = (1.0 + COMMIT) * acc[0, 0] / (TOK * EMB)
    quantize = jnp.swapaxes(q.reshape(x.shape), 1, -1)
    return quantize, diff, ind.reshape(x.shape[:-1])
